# baseline (speedup 1.0000x reference)
import jax
import jax.numpy as jnp
from jax.experimental import pallas as pl
from jax.experimental.pallas import tpu as pltpu

_K = 32
_S = 250
_HW = _S * _S
_NDROP = 4
_DB = 8


def _body(drop_ref, x_ref, o_ref):
    bd = pl.program_id(1)
    dcell = bd // (_K // _DB)
    hits = []
    for n in range(_NDROP):
        hits.append(drop_ref[n] // 64 == dcell)
    any_hit = hits[0] | hits[1] | hits[2] | hits[3]

    @pl.when(jnp.logical_not(any_hit))
    def _():
        o_ref[...] = x_ref[...]

    @pl.when(any_hit)
    def _():
        hw = jax.lax.broadcasted_iota(jnp.int32, (1, 1, _HW), 2)
        hcell = hw // _S // _K
        wcell = hw % _S // _K
        mask = None
        for n in range(_NDROP):
            p = drop_ref[n]
            m = hits[n] & ((p // 8) % 8 == hcell) & (p % 8 == wcell)
            mask = m if mask is None else mask | m
        o_ref[...] = jnp.where(mask, 0.0, x_ref[...])


def kernel(x, drop_idx):
    B = x.shape[0]
    xs = x.reshape(B, _S, _HW)
    nblk = (_S + _DB - 1) // _DB
    out = pl.pallas_call(
        _body,
        grid=(B, nblk),
        in_specs=[
            pl.BlockSpec(memory_space=pltpu.SMEM),
            pl.BlockSpec((1, _DB, _HW), lambda b, i: (b, i, 0)),
        ],
        out_specs=pl.BlockSpec((1, _DB, _HW), lambda b, i: (b, i, 0)),
        out_shape=jax.ShapeDtypeStruct((B, _S, _HW), jnp.float32),
        compiler_params=pltpu.CompilerParams(
            dimension_semantics=("parallel", "parallel"),
        ),
    )(drop_idx.astype(jnp.int32), xs)
    return out.reshape(x.shape)


# speedup vs baseline: 4.4715x; 4.4715x over previous
import functools
import jax
import jax.numpy as jnp
from jax import lax
from jax.experimental import pallas as pl
from jax.experimental.pallas import tpu as pltpu
from jax.experimental.pallas import tpu_sc as plsc

_K = 32
_S = 250
_B = 2
_ROWS = _B * _S * _S
_CH = 128
_NCH = _ROWS // _CH
_TAIL = _ROWS - _NCH * _CH
_NW = 32
_NDROP = 4
_SEGS = tuple(range(0, 240, 16)) + (_S - 16,)


def _emit_zero(buf, maskv, params, r0, rows):
    for n in range(_NDROP):
        pd, ph = params[n]
        for part in range(2):
            q = r0 // _S + part
            d = q % _S
            dtouch = (d >= pd * _K) & (d < pd * _K + _K)
            lo = jnp.maximum(q * _S + ph * _K, r0)
            hi = jnp.minimum(
                jnp.minimum(q * _S + ph * _K + _K, (q + 1) * _S), r0 + rows
            )
            llo = lo - r0
            lhi = hi - r0

            @pl.when(dtouch & (llo < lhi))
            def _(n=n, llo=llo, lhi=lhi):
                def rowbody(r, carry):
                    for off in _SEGS:
                        buf[r, pl.ds(off, 16)] = (
                            buf[r, pl.ds(off, 16)] * maskv[n, pl.ds(off, 16)]
                        )
                    return carry

                lax.fori_loop(llo, lhi, rowbody, 0)


@functools.partial(
    pl.kernel,
    out_type=jax.ShapeDtypeStruct((_ROWS, _S), jnp.float32),
    mesh=plsc.VectorSubcoreMesh(core_axis_name="c", subcore_axis_name="s"),
    scratch_types=[
        pltpu.VMEM((_CH, _S), jnp.float32),
        pltpu.VMEM((16,), jnp.int32),
        pltpu.VMEM((_NDROP, _S), jnp.float32),
    ],
)
def _sc_run(x_hbm, drop_hbm, masks_hbm, out_hbm, buf, idxv, maskv):
    wid = lax.axis_index("s") * 2 + lax.axis_index("c")
    pltpu.sync_copy(drop_hbm, idxv)
    pltpu.sync_copy(masks_hbm, maskv)
    v = idxv[...]
    params = []
    for n in range(_NDROP):
        p = v[n]
        params.append((p // 64, (p // 8) % 8))

    nk = -(-_NCH // _NW)

    def chunk_body(k, carry):
        chunk = wid + _NW * k

        @pl.when(chunk < _NCH)
        def _():
            r0 = chunk * _CH
            pltpu.sync_copy(x_hbm.at[pl.ds(r0, _CH)], buf)
            _emit_zero(buf, maskv, params, r0, _CH)
            pltpu.sync_copy(buf, out_hbm.at[pl.ds(r0, _CH)])

        return carry

    lax.fori_loop(0, nk, chunk_body, 0)

    @pl.when(wid == 16)
    def _():
        r0 = _NCH * _CH
        tbuf = buf.at[pl.ds(0, _TAIL)]
        pltpu.sync_copy(x_hbm.at[pl.ds(r0, _TAIL)], tbuf)
        _emit_zero(buf, maskv, params, r0, _TAIL)
        pltpu.sync_copy(tbuf, out_hbm.at[pl.ds(r0, _TAIL)])


def kernel(x, drop_idx):
    di = drop_idx.astype(jnp.int32)
    x2 = x.reshape(_ROWS, _S)
    dv = jnp.zeros((16,), jnp.int32).at[:_NDROP].set(di)
    w = jnp.arange(_S, dtype=jnp.int32)[None, :]
    pw = (di % 8)[:, None]
    masks = jnp.where((w >= pw * _K) & (w < pw * _K + _K), 0.0, 1.0).astype(
        jnp.float32
    )
    out = _sc_run(x2, dv, masks)
    return out.reshape(x.shape)


# speedup vs baseline: 13.5445x; 3.0290x over previous
import jax
import jax.numpy as jnp
from jax.experimental import pallas as pl
from jax.experimental.pallas import tpu as pltpu

_K = 32
_S = 250
_NDROP = 4


def _body(drop_ref, x_ref, o_ref):
    bd = pl.program_id(1)
    bh = pl.program_id(2)
    hits = []
    for n in range(_NDROP):
        p = drop_ref[n]
        hits.append((p // 64 == bd) & ((p // 8) % 8 == bh))
    any_hit = hits[0] | hits[1] | hits[2] | hits[3]

    @pl.when(jnp.logical_not(any_hit))
    def _():
        o_ref[...] = x_ref[...]

    @pl.when(any_hit)
    def _():
        wp = jax.lax.broadcasted_iota(jnp.int32, (1, 1, 1, _S), 3) // _K
        mask = None
        for n in range(_NDROP):
            m = hits[n] & (drop_ref[n] % 8 == wp)
            mask = m if mask is None else mask | m
        o_ref[...] = jnp.where(mask, 0.0, x_ref[...])


def kernel(x, drop_idx):
    B = x.shape[0]
    xs = x.reshape(B, _S, _S, _S)
    nblk = (_S + _K - 1) // _K
    out = pl.pallas_call(
        _body,
        grid=(B, nblk, nblk),
        in_specs=[
            pl.BlockSpec(memory_space=pltpu.SMEM),
            pl.BlockSpec((1, _K, _K, _S), lambda b, i, j: (b, i, j, 0)),
        ],
        out_specs=pl.BlockSpec((1, _K, _K, _S), lambda b, i, j: (b, i, j, 0)),
        out_shape=jax.ShapeDtypeStruct((B, _S, _S, _S), jnp.float32),
        compiler_params=pltpu.CompilerParams(
            dimension_semantics=("parallel", "parallel", "parallel"),
        ),
    )(drop_idx.astype(jnp.int32), xs)
    return out.reshape(x.shape)


# speedup vs baseline: 13.8240x; 1.0206x over previous
import functools
import jax
import jax.numpy as jnp
from jax import lax
from jax.experimental import pallas as pl
from jax.experimental.pallas import tpu as pltpu
from jax.experimental.pallas import tpu_sc as plsc

_K = 32
_S = 250
_B = 2
_NPL = _B * _S
_NW = 32
_NDROP = 4
_SEGS = tuple(range(0, 240, 16)) + (_S - 16,)


@functools.partial(
    pl.kernel,
    out_type=jax.ShapeDtypeStruct((_B, _S, _S, _S), jnp.float32),
    mesh=plsc.VectorSubcoreMesh(core_axis_name="c", subcore_axis_name="s"),
    scratch_types=[
        pltpu.VMEM((_S, _S), jnp.float32),
        pltpu.VMEM((16,), jnp.int32),
        pltpu.VMEM((_NDROP, _S), jnp.float32),
    ],
)
def _sc_run(x_hbm, drop_hbm, masks_hbm, out_hbm, buf, idxv, maskv):
    wid = lax.axis_index("s") * 2 + lax.axis_index("c")
    pltpu.sync_copy(drop_hbm, idxv)
    pltpu.sync_copy(masks_hbm, maskv)
    v = idxv[...]
    params = []
    for n in range(_NDROP):
        p = v[n]
        params.append((p // 64, (p // 8) % 8))

    def plane_body(k, carry):
        pidx = wid + _NW * k

        @pl.when(pidx < _NPL)
        def _():
            b = pidx // _S
            dd = pidx % _S
            pltpu.sync_copy(x_hbm.at[b, dd], buf)
            for n in range(_NDROP):
                pd, ph = params[n]
                dtouch = (dd >= pd * _K) & (dd < pd * _K + _K)

                @pl.when(dtouch)
                def _(n=n, ph=ph):
                    def rowbody(r, c):
                        for off in _SEGS:
                            buf[r, pl.ds(off, 16)] = (
                                buf[r, pl.ds(off, 16)]
                                * maskv[n, pl.ds(off, 16)]
                            )
                        return c

                    lax.fori_loop(
                        ph * _K, jnp.minimum(_S, ph * _K + _K), rowbody, 0
                    )

            pltpu.sync_copy(buf, out_hbm.at[b, dd])

        return carry

    lax.fori_loop(0, -(-_NPL // _NW), plane_body, 0)


def kernel(x, drop_idx):
    di = drop_idx.astype(jnp.int32)
    x4 = x.reshape(_B, _S, _S, _S)
    dv = jnp.zeros((16,), jnp.int32).at[:_NDROP].set(di)
    w = jnp.arange(_S, dtype=jnp.int32)[None, :]
    pw = (di % 8)[:, None]
    masks = jnp.where((w >= pw * _K) & (w < pw * _K + _K), 0.0, 1.0).astype(
        jnp.float32
    )
    out = _sc_run(x4, dv, masks)
    return out.reshape(x.shape)
